# baseline (speedup 1.0000x reference)
import jax
import jax.numpy as jnp
from jax import lax
from jax.experimental import pallas as pl
from jax.experimental.pallas import tpu as pltpu
from jax.experimental.pallas import tpu_sc as plsc

C = 1000
B = 16384
BB = 512
NB = B // BB

NSUB = 16
IDX_PER_SUB = B // NSUB
CC_PAD = 1000064
ZSLICE = CC_PAD // NSUB
OSLICE = 62496


def _argmax_body(pred_ref, tgt_ref, out_ref):
    x = pred_ref[...]
    mx = jnp.max(x, axis=1, keepdims=True)
    col = jax.lax.broadcasted_iota(jnp.int32, x.shape, 1)
    p = jnp.min(jnp.where(x == mx, col, C), axis=1)
    out_ref[0, 0, :] = tgt_ref[0, 0, :] * C + p


def _flat_indices(prediction, target):
    tgt3 = target.reshape(NB, 1, BB)
    out = pl.pallas_call(
        _argmax_body,
        grid=(NB,),
        in_specs=[
            pl.BlockSpec((BB, C), lambda i: (i, 0)),
            pl.BlockSpec((1, 1, BB), lambda i: (i, 0, 0)),
        ],
        out_specs=pl.BlockSpec((1, 1, BB), lambda i: (i, 0, 0)),
        out_shape=jax.ShapeDtypeStruct((NB, 1, BB), jnp.int32),
        compiler_params=pltpu.CompilerParams(
            dimension_semantics=("arbitrary",),
        ),
    )(prediction, tgt3)
    return out.reshape(B)


LAST_SLICE = C * C - (NSUB - 1) * ZSLICE


def _sc_histogram(flat_idx):
    mesh = plsc.VectorSubcoreMesh(core_axis_name="c", subcore_axis_name="s")

    @pl.kernel(
        out_type=jax.ShapeDtypeStruct((C * C,), jnp.float32),
        mesh=mesh,
        scratch_types=[
            pltpu.VMEM((IDX_PER_SUB,), jnp.int32),
            pltpu.VMEM((IDX_PER_SUB,), jnp.float32),
            pltpu.VMEM((ZSLICE,), jnp.float32),
            pltpu.VMEM_SHARED((CC_PAD,), jnp.float32),
        ],
    )
    def hist_kernel(idx_hbm, out_hbm, idx_v, ones_v, buf_v, acc):
        cid = lax.axis_index("c")
        sid = lax.axis_index("s")

        @pl.when(cid == 0)
        def _():
            @pl.loop(0, ZSLICE, step=16)
            def _(i):
                buf_v.at[pl.ds(i, 16)][...] = jnp.zeros((16,), jnp.float32)

            pltpu.sync_copy(buf_v, acc.at[pl.ds(sid * ZSLICE, ZSLICE)])

            @pl.loop(0, IDX_PER_SUB, step=16)
            def _(i):
                ones_v.at[pl.ds(i, 16)][...] = jnp.full((16,), 1.0, jnp.float32)

            pltpu.sync_copy(idx_hbm.at[pl.ds(sid * IDX_PER_SUB, IDX_PER_SUB)], idx_v)
            plsc.subcore_barrier()
            pltpu.sync_copy(ones_v, acc.at[idx_v], add=True)
            plsc.subcore_barrier()

            @pl.when(sid < NSUB - 1)
            def _():
                pltpu.sync_copy(acc.at[pl.ds(sid * ZSLICE, ZSLICE)], buf_v)
                pltpu.sync_copy(
                    buf_v, out_hbm.at[pl.ds(sid * ZSLICE, ZSLICE)]
                )

            @pl.when(sid == NSUB - 1)
            def _():
                pltpu.sync_copy(
                    acc.at[pl.ds(sid * ZSLICE, LAST_SLICE)],
                    buf_v.at[pl.ds(0, LAST_SLICE)],
                )
                pltpu.sync_copy(
                    buf_v.at[pl.ds(0, LAST_SLICE)],
                    out_hbm.at[pl.ds(sid * ZSLICE, LAST_SLICE)],
                )

    return hist_kernel(flat_idx)


def kernel(prediction, target):
    flat_idx = _flat_indices(prediction, target)
    cm_flat = _sc_histogram(flat_idx)
    return cm_flat.reshape(C, C)


# speedup vs baseline: 2.0093x; 2.0093x over previous
import dataclasses

import jax
import jax.numpy as jnp
from jax import lax
from jax.experimental import pallas as pl
from jax.experimental.pallas import tpu as pltpu
from jax.experimental.pallas import tpu_sc as plsc

C = 1000
B = 16384
BB = 512
NB = B // BB

NSUB = 16
NCORE = 2
NTILE = NCORE * NSUB
SHIFT = 15
BINS = 1 << SHIFT
BINS_30 = C * C - 30 * BINS


def _argmax_body(pred_ref, tgt_ref, out_ref):
    x = pred_ref[...]
    mx = jnp.max(x, axis=0, keepdims=True)
    row = jax.lax.broadcasted_iota(jnp.int32, x.shape, 0)
    p = jnp.min(jnp.where(x == mx, row, C), axis=0)
    out_ref[0, 0, :] = tgt_ref[0, 0, :] * C + p


def _flat_indices(prediction, target):
    pred_t = prediction.T
    tgt3 = target.reshape(NB, 1, BB)
    out = pl.pallas_call(
        _argmax_body,
        grid=(NB,),
        in_specs=[
            pl.BlockSpec((C, BB), lambda i: (0, i)),
            pl.BlockSpec((1, 1, BB), lambda i: (i, 0, 0)),
        ],
        out_specs=pl.BlockSpec((1, 1, BB), lambda i: (i, 0, 0)),
        out_shape=jax.ShapeDtypeStruct((NB, 1, BB), jnp.int32),
        compiler_params=pltpu.CompilerParams(
            dimension_semantics=("parallel",),
        ),
    )(pred_t, tgt3)
    return out.reshape(B)


def _sc_histogram(flat_idx):
    mesh = plsc.VectorSubcoreMesh(core_axis_name="c", subcore_axis_name="s")
    cp = pltpu.CompilerParams()
    if "needs_layout_passes" in pltpu.CompilerParams.__dataclass_fields__:
        cp = dataclasses.replace(cp, needs_layout_passes=False)

    @pl.kernel(
        compiler_params=cp,
        out_type=jax.ShapeDtypeStruct((C * C,), jnp.float32),
        mesh=mesh,
        scratch_types=[
            pltpu.VMEM((B,), jnp.int32),
            pltpu.VMEM((BINS,), jnp.float32),
        ],
    )
    def hist_kernel(idx_hbm, out_hbm, idx_v, bins_v):
        cid = lax.axis_index("c")
        sid = lax.axis_index("s")
        wid = cid * NSUB + sid

        zeros = jnp.zeros((16,), jnp.float32)

        @pl.loop(0, BINS, step=64)
        def _(i):
            for k in range(4):
                bins_v.at[pl.ds(i + 16 * k, 16)][...] = zeros

        pltpu.sync_copy(idx_hbm, idx_v)

        ones = jnp.full((16,), 1.0, jnp.float32)

        @pl.loop(0, B, step=64)
        def _(i):
            for k in range(4):
                v = idx_v.at[pl.ds(i + 16 * k, 16)][...]
                m = (v >> SHIFT) == wid
                local = v & (BINS - 1)
                plsc.addupdate_scatter(bins_v, [local], ones, mask=m)

        @pl.when(wid < 30)
        def _():
            pltpu.sync_copy(
                bins_v, out_hbm.at[pl.ds(wid * BINS, BINS)]
            )

        @pl.when(wid == 30)
        def _():
            pltpu.sync_copy(
                bins_v.at[pl.ds(0, BINS_30)],
                out_hbm.at[pl.ds(30 * BINS, BINS_30)],
            )

    return hist_kernel(flat_idx)


def kernel(prediction, target):
    flat_idx = _flat_indices(prediction, target)
    cm_flat = _sc_histogram(flat_idx)
    return cm_flat.reshape(C, C)


# speedup vs baseline: 2.3945x; 1.1917x over previous
import dataclasses

import jax
import jax.numpy as jnp
from jax import lax
from jax.experimental import pallas as pl
from jax.experimental.pallas import tpu as pltpu
from jax.experimental.pallas import tpu_sc as plsc

C = 1000
B = 16384
BB = 2048
NB = B // BB

NSUB = 16
NCORE = 2
NTILE = NCORE * NSUB
SHIFT = 15
BINS = 1 << SHIFT
BINS_30 = C * C - 30 * BINS


def _argmax_body(pred_ref, tgt_ref, out_ref):
    x = pred_ref[...]
    mx = jnp.max(x, axis=0, keepdims=True)
    row = jax.lax.broadcasted_iota(jnp.int32, x.shape, 0)
    p = jnp.min(jnp.where(x == mx, row, C), axis=0)
    out_ref[0, 0, :] = tgt_ref[0, 0, :] * C + p


def _flat_indices(prediction, target):
    pred_t = prediction.T
    tgt3 = target.reshape(NB, 1, BB)
    out = pl.pallas_call(
        _argmax_body,
        grid=(NB,),
        in_specs=[
            pl.BlockSpec((C, BB), lambda i: (0, i)),
            pl.BlockSpec((1, 1, BB), lambda i: (i, 0, 0)),
        ],
        out_specs=pl.BlockSpec((1, 1, BB), lambda i: (i, 0, 0)),
        out_shape=jax.ShapeDtypeStruct((NB, 1, BB), jnp.int32),
        compiler_params=pltpu.CompilerParams(
            dimension_semantics=("parallel",),
        ),
    )(pred_t, tgt3)
    return out.reshape(B)


def _sc_histogram(flat_idx):
    mesh = plsc.VectorSubcoreMesh(core_axis_name="c", subcore_axis_name="s")
    cp = pltpu.CompilerParams()
    if "needs_layout_passes" in pltpu.CompilerParams.__dataclass_fields__:
        cp = dataclasses.replace(cp, needs_layout_passes=False)

    @pl.kernel(
        compiler_params=cp,
        out_type=jax.ShapeDtypeStruct((C * C,), jnp.float32),
        mesh=mesh,
        scratch_types=[
            pltpu.VMEM((B,), jnp.int32),
            pltpu.VMEM((BINS,), jnp.float32),
        ],
    )
    def hist_kernel(idx_hbm, out_hbm, idx_v, bins_v):
        cid = lax.axis_index("c")
        sid = lax.axis_index("s")
        wid = cid * NSUB + sid

        zeros = jnp.zeros((16,), jnp.float32)

        @pl.loop(0, BINS, step=64)
        def _(i):
            for k in range(4):
                bins_v.at[pl.ds(i + 16 * k, 16)][...] = zeros

        pltpu.sync_copy(idx_hbm, idx_v)

        ones = jnp.full((16,), 1.0, jnp.float32)

        @pl.loop(0, B, step=64)
        def _(i):
            for k in range(4):
                v = idx_v.at[pl.ds(i + 16 * k, 16)][...]
                m = (v >> SHIFT) == wid
                local = v & (BINS - 1)
                plsc.addupdate_scatter(bins_v, [local], ones, mask=m)

        @pl.when(wid < 30)
        def _():
            pltpu.sync_copy(
                bins_v, out_hbm.at[pl.ds(wid * BINS, BINS)]
            )

        @pl.when(wid == 30)
        def _():
            pltpu.sync_copy(
                bins_v.at[pl.ds(0, BINS_30)],
                out_hbm.at[pl.ds(30 * BINS, BINS_30)],
            )

    return hist_kernel(flat_idx)


def kernel(prediction, target):
    flat_idx = _flat_indices(prediction, target)
    cm_flat = _sc_histogram(flat_idx)
    return cm_flat.reshape(C, C)
